# baseline (speedup 1.0000x reference)
import math

import jax
import jax.numpy as jnp
import numpy as np
from jax.experimental import pallas as pl

_NUM_VIEWS = 24
_NUM_INPUT = 16
_NUM_TARGET = 8


def _split_indices(total_views, num_input_views, num_target_views):
    g = math.gcd(num_input_views, num_target_views)
    group_size = total_views // g
    in_per_group = num_input_views // g
    tar_per_group = num_target_views // g
    input_indices = []
    target_indices = []
    for group_idx in range(g):
        start = group_idx * group_size
        block = list(range(start, start + group_size))
        input_indices.extend(block[:in_per_group])
        target_indices.extend(block[in_per_group:in_per_group + tar_per_group])
    input_indices = np.sort(np.array(input_indices, dtype=np.int32))
    target_indices = np.sort(np.array(target_indices, dtype=np.int32))
    return input_indices, target_indices


def _split_kernel(img_ref, in_ref, tar_ref):
    blk = img_ref[...]
    in_ref[...] = blk[:, :2]
    tar_ref[...] = blk[:, 2:3]


def kernel(image):
    B, V, C, H, W = image.shape
    G = V // 3
    rows = B * G
    flat = image.reshape(rows, 3, C * H, W)

    input_flat, target_flat = pl.pallas_call(
        _split_kernel,
        grid=(rows,),
        in_specs=[pl.BlockSpec((1, 3, C * H, W), lambda r: (r, 0, 0, 0))],
        out_specs=[
            pl.BlockSpec((1, 2, C * H, W), lambda r: (r, 0, 0, 0)),
            pl.BlockSpec((1, 1, C * H, W), lambda r: (r, 0, 0, 0)),
        ],
        out_shape=[
            jax.ShapeDtypeStruct((rows, 2, C * H, W), image.dtype),
            jax.ShapeDtypeStruct((rows, 1, C * H, W), image.dtype),
        ],
    )(flat)

    input_image = input_flat.reshape(B, 2 * G, C, H, W)
    target_image = target_flat.reshape(B, G, C, H, W)

    ii, ti = _split_indices(_NUM_VIEWS, _NUM_INPUT, _NUM_TARGET)
    input_pattern = jnp.tile(jnp.asarray(ii)[None, :], (B, 1))
    target_pattern = jnp.tile(jnp.asarray(ti)[None, :], (B, 1))
    return (input_image, target_image, input_pattern, target_pattern)
